# baseline (speedup 1.0000x reference)
import functools
import math

import jax
import jax.numpy as jnp
from jax.experimental import pallas as pl
from jax.experimental.pallas import tpu as pltpu

_EPS = 1e-05
_BETA = 0.7
_LOG2E = math.log2(math.e)
_LN2 = math.log(2.0)
_LOG2_EPS = math.log2(_EPS)
_LOG2_1M_EPS = math.log1p(-_EPS) / _LN2


def _partial_body(c_ref, x_ref, t_ref, out_ref):
    x = x_ref[...]
    tgt = t_ref[...]
    m = (c_ref[...] == 0).astype(jnp.float32)
    cm = (1.0 - _BETA) * m

    X = x * _LOG2E
    e2 = jnp.exp2(jnp.minimum(X, -X))
    l2 = jnp.log2(1.0 + e2)
    lp2 = jnp.clip(jnp.minimum(X, 0.0) - l2, _LOG2_EPS, _LOG2_1M_EPS)
    lq2 = jnp.clip(lp2 - X, _LOG2_EPS, _LOG2_1M_EPS)
    d2 = lp2 - lq2
    pred = jnp.exp2(lp2)
    t_eff = tgt + cm * (pred - tgt)
    bce2 = lq2 + t_eff * d2

    out_ref[0, 0:1, :] = jnp.sum(bce2 * m, axis=0, keepdims=True)
    out_ref[0, 1:2, :] = jnp.sum(bce2, axis=0, keepdims=True)


def _final_body(p_ref, c_ref, out_ref, *, bs, o, rows):
    noisy_sum = -_LN2 * jnp.sum(p_ref[:, 0, :])
    cur_sum = -_LN2 * jnp.sum(p_ref[:, 1, :]) - noisy_sum
    nl = jnp.sum((c_ref[...] == 0).astype(jnp.float32))
    cl = float(rows) - nl
    noisy_loss = noisy_sum / (nl * float(o)) * (nl / float(bs))
    curated_loss = cur_sum / (cl * float(o)) * (cl / float(bs))
    out_ref[0] = noisy_loss * 0.5 + curated_loss * 0.5
    out_ref[1] = noisy_loss
    out_ref[2] = curated_loss


def kernel(output, target, clean):
    bs, seq, o = target.shape
    rows = bs * seq
    x = output.reshape(rows, o)
    t = target.reshape(rows, o)
    c = clean.reshape(rows, 1)
    block_rows = 2048
    grid = rows // block_rows
    partials = pl.pallas_call(
        _partial_body,
        grid=(grid,),
        in_specs=[
            pl.BlockSpec((block_rows, 1), lambda i: (i, 0)),
            pl.BlockSpec((block_rows, o), lambda i: (i, 0)),
            pl.BlockSpec((block_rows, o), lambda i: (i, 0)),
        ],
        out_specs=pl.BlockSpec((1, 2, o), lambda i: (i, 0, 0)),
        out_shape=jax.ShapeDtypeStruct((grid, 2, o), jnp.float32),
        compiler_params=pltpu.CompilerParams(
            dimension_semantics=("parallel",)),
    )(c, x, t)
    final = functools.partial(_final_body, bs=bs, o=o, rows=rows)
    out = pl.pallas_call(
        final,
        out_specs=pl.BlockSpec(memory_space=pltpu.SMEM),
        out_shape=jax.ShapeDtypeStruct((3,), jnp.float32),
    )(partials, c)
    return (out[0], out[1], out[2])


# speedup vs baseline: 1.2104x; 1.2104x over previous
import functools
import math

import jax
import jax.numpy as jnp
from jax.experimental import pallas as pl
from jax.experimental.pallas import tpu as pltpu

_EPS = 1e-05
_BETA = 0.7
_LOG2E = math.log2(math.e)
_LN2 = math.log(2.0)
_XLO = (math.log(_EPS) - math.log1p(-_EPS)) / _LN2
_XHI = -_XLO


def _loss_body(c_ref, x_ref, t_ref, out_ref, acc_ref, cnt_ref, *, bs, o, rows):
    i = pl.program_id(0)

    @pl.when(i == 0)
    def _init():
        acc_ref[...] = jnp.zeros_like(acc_ref)
        cnt_ref[0] = 0.0

    x = x_ref[...]
    tgt = t_ref[...]
    m = (c_ref[...] == 0).astype(jnp.float32)
    cm = (1.0 - _BETA) * m

    Xc = jnp.clip(x * _LOG2E, _XLO, _XHI)
    e2 = jnp.exp2(jnp.minimum(Xc, -Xc))
    l2 = jnp.log2(1.0 + e2)
    lp2 = jnp.minimum(Xc, 0.0) - l2
    pred = jnp.exp2(lp2)
    g = (1.0 - tgt) - cm * (pred - tgt)
    bce2 = lp2 - g * Xc

    acc_ref[0:1, :] += jnp.sum(bce2 * m, axis=0, keepdims=True)
    acc_ref[1:2, :] += jnp.sum(bce2, axis=0, keepdims=True)
    cnt_ref[0] += jnp.sum(m)

    @pl.when(i == pl.num_programs(0) - 1)
    def _finish():
        noisy_sum = -_LN2 * jnp.sum(acc_ref[0:1, :])
        cur_sum = -_LN2 * jnp.sum(acc_ref[1:2, :]) - noisy_sum
        nl = cnt_ref[0]
        cl = float(rows) - nl
        noisy_loss = noisy_sum / (nl * float(o)) * (nl / float(bs))
        curated_loss = cur_sum / (cl * float(o)) * (cl / float(bs))
        out_ref[0] = noisy_loss * 0.5 + curated_loss * 0.5
        out_ref[1] = noisy_loss
        out_ref[2] = curated_loss


def kernel(output, target, clean):
    bs, seq, o = target.shape
    rows = bs * seq
    x = output.reshape(rows, o)
    t = target.reshape(rows, o)
    c = clean.reshape(rows, 1)
    block_rows = 2048
    body = functools.partial(_loss_body, bs=bs, o=o, rows=rows)
    out = pl.pallas_call(
        body,
        grid=(rows // block_rows,),
        in_specs=[
            pl.BlockSpec((block_rows, 1), lambda i: (i, 0)),
            pl.BlockSpec((block_rows, o), lambda i: (i, 0)),
            pl.BlockSpec((block_rows, o), lambda i: (i, 0)),
        ],
        out_specs=pl.BlockSpec(memory_space=pltpu.SMEM),
        out_shape=jax.ShapeDtypeStruct((3,), jnp.float32),
        scratch_shapes=[
            pltpu.VMEM((2, o), jnp.float32),
            pltpu.SMEM((1,), jnp.float32),
        ],
    )(c, x, t)
    return (out[0], out[1], out[2])


# speedup vs baseline: 1.5082x; 1.2461x over previous
import functools
import math

import jax
import jax.numpy as jnp
from jax.experimental import pallas as pl
from jax.experimental.pallas import tpu as pltpu

_EPS = 1e-05
_BETA = 0.7
_LOG2E = math.log2(math.e)
_LN2 = math.log(2.0)
_XLO = (math.log(_EPS) - math.log1p(-_EPS)) / _LN2
_XHI = -_XLO


def _loss_body(c_ref, x_ref, t_ref, out_ref, acc_ref, cnt_ref, *, bs, o, rows):
    i = pl.program_id(0)

    @pl.when(i == 0)
    def _init():
        acc_ref[...] = jnp.zeros_like(acc_ref)
        cnt_ref[0] = 0.0

    x = x_ref[...]
    tgt = t_ref[...]
    m_row = (c_ref[0] == 0).astype(jnp.float32)

    Xc = jnp.clip(x * _LOG2E, _XLO, _XHI)
    e2 = jnp.exp2(jnp.minimum(Xc, -Xc))
    l2 = jnp.log2(1.0 + e2)
    lp2 = jnp.minimum(Xc, 0.0) - l2
    pred = jnp.exp2(lp2)
    base2 = lp2 - (1.0 - tgt) * Xc
    q = (pred - tgt) * Xc

    row_id = jax.lax.broadcasted_iota(jnp.int32, (8, m_row.shape[1]), 0)
    lhs = jnp.where(row_id == 0, 1.0, jnp.where(row_id == 1, m_row, 0.0))
    lhs_bf = lhs.astype(jnp.bfloat16)
    acc_ref[0:8, :] += jnp.dot(lhs_bf, base2.astype(jnp.bfloat16),
                               preferred_element_type=jnp.float32)
    acc_ref[8:16, :] += jnp.dot(lhs_bf, q.astype(jnp.bfloat16),
                                preferred_element_type=jnp.float32)
    cnt_ref[0] += jnp.sum(m_row)

    @pl.when(i == pl.num_programs(0) - 1)
    def _finish():
        sum_base_all = jnp.sum(acc_ref[0:1, :])
        sum_base_msk = jnp.sum(acc_ref[1:2, :])
        sum_q_msk = jnp.sum(acc_ref[9:10, :])
        noisy_sum = -_LN2 * (sum_base_msk + (1.0 - _BETA) * sum_q_msk)
        cur_sum = -_LN2 * (sum_base_all - sum_base_msk)
        nl = cnt_ref[0]
        cl = float(rows) - nl
        noisy_loss = noisy_sum / (nl * float(o)) * (nl / float(bs))
        curated_loss = cur_sum / (cl * float(o)) * (cl / float(bs))
        out_ref[0] = noisy_loss * 0.5 + curated_loss * 0.5
        out_ref[1] = noisy_loss
        out_ref[2] = curated_loss


def kernel(output, target, clean):
    bs, seq, o = target.shape
    rows = bs * seq
    x = output.reshape(rows, o)
    t = target.reshape(rows, o)
    block_rows = 2048
    grid = rows // block_rows
    c = clean.reshape(grid, 1, block_rows)
    body = functools.partial(_loss_body, bs=bs, o=o, rows=rows)
    out = pl.pallas_call(
        body,
        grid=(grid,),
        in_specs=[
            pl.BlockSpec((1, 1, block_rows), lambda i: (i, 0, 0)),
            pl.BlockSpec((block_rows, o), lambda i: (i, 0)),
            pl.BlockSpec((block_rows, o), lambda i: (i, 0)),
        ],
        out_specs=pl.BlockSpec(memory_space=pltpu.SMEM),
        out_shape=jax.ShapeDtypeStruct((3,), jnp.float32),
        scratch_shapes=[
            pltpu.VMEM((16, o), jnp.float32),
            pltpu.SMEM((1,), jnp.float32),
        ],
    )(c, x, t)
    return (out[0], out[1], out[2])
